# Initial kernel scaffold; baseline (speedup 1.0000x reference)
import jax
import jax.numpy as jnp
from jax.experimental import pallas as pl


def kernel(positions, scales, contact_points, contact_normals, contact_confidence):
    raise NotImplementedError("write your pallas kernel here")


# speedup vs baseline: 1.3986x; 1.3986x over previous
import functools

import jax
import jax.numpy as jnp
from jax import lax
from jax.experimental import pallas as pl
from jax.experimental.pallas import tpu as pltpu
from jax.experimental.pallas import tpu_sc as plsc

_NC = 2
_NS = 16
_L = 16
_NW = _NC * _NS


def _argmin_body(mu_ref, cpt_ref, out_ref, bmin_ref, barg_ref, *, tn, n_tiles):
    i = pl.program_id(0)
    mu = mu_ref[...]
    score = jnp.sum(mu * mu, axis=1, keepdims=True) - 2.0 * jnp.dot(
        mu, cpt_ref[...], preferred_element_type=jnp.float32
    )
    lmin = jnp.min(score, axis=0, keepdims=True)
    rows = lax.broadcasted_iota(jnp.int32, score.shape, 0) + i * tn
    big = jnp.int32(2**30)
    larg = jnp.min(jnp.where(score == lmin, rows, big), axis=0, keepdims=True)

    @pl.when(i == 0)
    def _():
        bmin_ref[...] = lmin
        barg_ref[...] = larg

    @pl.when(i > 0)
    def _():
        better = lmin < bmin_ref[...]
        bmin_ref[...] = jnp.where(better, lmin, bmin_ref[...])
        barg_ref[...] = jnp.where(better, larg, barg_ref[...])

    @pl.when(i == n_tiles - 1)
    def _():
        out_ref[...] = barg_ref[...]


def _nearest_idx(positions, contact_points, tn):
    n, p = positions.shape[0], contact_points.shape[0]
    n_tiles = n // tn
    mu8 = jnp.pad(positions, ((0, 0), (0, 5)))
    cp8t = jnp.pad(contact_points, ((0, 0), (0, 5))).T
    nn = pl.pallas_call(
        functools.partial(_argmin_body, tn=tn, n_tiles=n_tiles),
        grid=(n_tiles,),
        in_specs=[
            pl.BlockSpec((tn, 8), lambda i: (i, 0)),
            pl.BlockSpec((8, p), lambda i: (0, 0)),
        ],
        out_specs=pl.BlockSpec((1, p), lambda i: (0, 0)),
        out_shape=jax.ShapeDtypeStruct((1, p), jnp.int32),
        scratch_shapes=[
            pltpu.VMEM((1, p), jnp.float32),
            pltpu.VMEM((1, p), jnp.int32),
        ],
    )(mu8, cp8t)
    return nn.reshape(p)


def _sc_residual_body(mux_hbm, muy_hbm, muz_hbm, scx_hbm, scy_hbm, scz_hbm,
                      idx_hbm, cpx_hbm, cpy_hbm, cpz_hbm, w_hbm, out_hbm,
                      idx_v, gmux_v, gmuy_v, gmuz_v, gscx_v, gscy_v, gscz_v,
                      cpx_v, cpy_v, cpz_v, w_v, acc_v, sem, *, ppw):
    wid = lax.axis_index("s") * _NC + lax.axis_index("c")
    base = wid * ppw
    pltpu.sync_copy(idx_hbm.at[pl.ds(base, ppw)], idx_v)
    pltpu.sync_copy(cpx_hbm.at[pl.ds(base, ppw)], cpx_v)
    pltpu.sync_copy(cpy_hbm.at[pl.ds(base, ppw)], cpy_v)
    pltpu.sync_copy(cpz_hbm.at[pl.ds(base, ppw)], cpz_v)
    pltpu.sync_copy(w_hbm.at[pl.ds(base, ppw)], w_v)
    copies = [
        pltpu.async_copy(mux_hbm.at[idx_v], gmux_v, sem),
        pltpu.async_copy(muy_hbm.at[idx_v], gmuy_v, sem),
        pltpu.async_copy(muz_hbm.at[idx_v], gmuz_v, sem),
        pltpu.async_copy(scx_hbm.at[idx_v], gscx_v, sem),
        pltpu.async_copy(scy_hbm.at[idx_v], gscy_v, sem),
        pltpu.async_copy(scz_hbm.at[idx_v], gscz_v, sem),
    ]
    for c in copies:
        c.wait()

    acc = jnp.zeros((_L,), jnp.float32)
    for g in range(ppw // _L):
        sl = pl.ds(g * _L, _L)
        m2 = jnp.zeros((_L,), jnp.float32)
        for cp_v, gmu_v, gsc_v in (
            (cpx_v, gmux_v, gscx_v),
            (cpy_v, gmuy_v, gscy_v),
            (cpz_v, gmuz_v, gscz_v),
        ):
            delta = (cp_v[sl] - gmu_v[sl]) / (jnp.exp(gsc_v[sl]) + 1e-6)
            m2 = m2 + delta * delta
        seed = (lax.bitcast_convert_type(m2, jnp.int32) >> 1) + jnp.int32(
            0x1FBD1DF5
        )
        y = lax.bitcast_convert_type(seed, jnp.float32)
        for _ in range(3):
            y = 0.5 * (y + m2 / y)
        r = y - 1.0
        wv = jnp.clip(w_v[sl], 0.0, 1.0)
        acc = acc + r * r * wv
    acc_v[...] = acc
    pltpu.sync_copy(acc_v, out_hbm.at[wid])


def _sc_residual(positions, scales, nn_idx, contact_points,
                 contact_confidence):
    p = contact_points.shape[0]
    ppw = p // _NW
    mesh = plsc.VectorSubcoreMesh(core_axis_name="c", subcore_axis_name="s")
    f32 = jnp.float32
    run = pl.kernel(
        functools.partial(_sc_residual_body, ppw=ppw),
        out_type=jax.ShapeDtypeStruct((_NW, _L), f32),
        mesh=mesh,
        scratch_types=[
            pltpu.VMEM((ppw,), jnp.int32),
            pltpu.VMEM((ppw,), f32),
            pltpu.VMEM((ppw,), f32),
            pltpu.VMEM((ppw,), f32),
            pltpu.VMEM((ppw,), f32),
            pltpu.VMEM((ppw,), f32),
            pltpu.VMEM((ppw,), f32),
            pltpu.VMEM((ppw,), f32),
            pltpu.VMEM((ppw,), f32),
            pltpu.VMEM((ppw,), f32),
            pltpu.VMEM((ppw,), f32),
            pltpu.VMEM((_L,), f32),
            pltpu.SemaphoreType.DMA,
        ],
    )
    return run(
        positions[:, 0],
        positions[:, 1],
        positions[:, 2],
        scales[:, 0],
        scales[:, 1],
        scales[:, 2],
        nn_idx,
        contact_points[:, 0],
        contact_points[:, 1],
        contact_points[:, 2],
        contact_confidence,
    )


def kernel(positions, scales, contact_points, contact_normals,
           contact_confidence):
    del contact_normals
    n = positions.shape[0]
    p = contact_points.shape[0]

    nn_idx = _nearest_idx(positions, contact_points, tn=512)
    partials = _sc_residual(positions, scales, nn_idx, contact_points,
                            contact_confidence)
    return jnp.sum(partials) / jnp.float32(p)


# speedup vs baseline: 1.5475x; 1.1065x over previous
import functools

import jax
import jax.numpy as jnp
from jax import lax
from jax.experimental import pallas as pl
from jax.experimental.pallas import tpu as pltpu
from jax.experimental.pallas import tpu_sc as plsc

_NC = 2
_NS = 16
_L = 16
_NW = _NC * _NS


def _argmin_body(mu_ref, cpt_ref, out_ref, bmin_ref, barg_ref, *, tn, n_tiles):
    i = pl.program_id(0)
    score = jnp.dot(
        mu_ref[...], cpt_ref[...], preferred_element_type=jnp.float32
    )
    lmin = jnp.min(score, axis=0, keepdims=True)
    rowsf = lax.broadcasted_iota(jnp.int32, score.shape, 0).astype(
        jnp.float32
    )
    inf = jnp.float32(jnp.inf)
    larg = jnp.min(
        jnp.where(score == lmin, rowsf, inf), axis=0, keepdims=True
    ) + jnp.float32(i * tn)

    @pl.when(i == 0)
    def _():
        bmin_ref[...] = lmin
        barg_ref[...] = larg

    @pl.when(i > 0)
    def _():
        better = lmin < bmin_ref[...]
        bmin_ref[...] = jnp.where(better, lmin, bmin_ref[...])
        barg_ref[...] = jnp.where(better, larg, barg_ref[...])

    @pl.when(i == n_tiles - 1)
    def _():
        out_ref[...] = barg_ref[...].astype(jnp.int32)


def _nearest_idx(positions, contact_points, tn):
    n, p = positions.shape[0], contact_points.shape[0]
    n_tiles = n // tn
    mu_sq = jnp.sum(positions * positions, axis=1, keepdims=True)
    mu8 = jnp.pad(
        jnp.concatenate([-2.0 * positions, mu_sq], axis=1), ((0, 0), (0, 4))
    )
    cp8t = jnp.pad(
        jnp.concatenate(
            [contact_points, jnp.ones((p, 1), jnp.float32)], axis=1
        ),
        ((0, 0), (0, 4)),
    ).T
    nn = pl.pallas_call(
        functools.partial(_argmin_body, tn=tn, n_tiles=n_tiles),
        grid=(n_tiles,),
        in_specs=[
            pl.BlockSpec((tn, 8), lambda i: (i, 0)),
            pl.BlockSpec((8, p), lambda i: (0, 0)),
        ],
        out_specs=pl.BlockSpec((1, p), lambda i: (0, 0)),
        out_shape=jax.ShapeDtypeStruct((1, p), jnp.int32),
        scratch_shapes=[
            pltpu.VMEM((1, p), jnp.float32),
            pltpu.VMEM((1, p), jnp.float32),
        ],
    )(mu8, cp8t)
    return nn.reshape(p)


def _sc_residual_body(mux_hbm, muy_hbm, muz_hbm, scx_hbm, scy_hbm, scz_hbm,
                      idx_hbm, cpx_hbm, cpy_hbm, cpz_hbm, w_hbm, out_hbm,
                      idx_v, gmux_v, gmuy_v, gmuz_v, gscx_v, gscy_v, gscz_v,
                      cpx_v, cpy_v, cpz_v, w_v, acc_v, sem, *, ppw):
    wid = lax.axis_index("s") * _NC + lax.axis_index("c")
    base = wid * ppw
    pltpu.sync_copy(idx_hbm.at[pl.ds(base, ppw)], idx_v)
    pltpu.sync_copy(cpx_hbm.at[pl.ds(base, ppw)], cpx_v)
    pltpu.sync_copy(cpy_hbm.at[pl.ds(base, ppw)], cpy_v)
    pltpu.sync_copy(cpz_hbm.at[pl.ds(base, ppw)], cpz_v)
    pltpu.sync_copy(w_hbm.at[pl.ds(base, ppw)], w_v)
    copies = [
        pltpu.async_copy(mux_hbm.at[idx_v], gmux_v, sem),
        pltpu.async_copy(muy_hbm.at[idx_v], gmuy_v, sem),
        pltpu.async_copy(muz_hbm.at[idx_v], gmuz_v, sem),
        pltpu.async_copy(scx_hbm.at[idx_v], gscx_v, sem),
        pltpu.async_copy(scy_hbm.at[idx_v], gscy_v, sem),
        pltpu.async_copy(scz_hbm.at[idx_v], gscz_v, sem),
    ]
    for c in copies:
        c.wait()

    acc = jnp.zeros((_L,), jnp.float32)
    for g in range(ppw // _L):
        sl = pl.ds(g * _L, _L)
        m2 = jnp.zeros((_L,), jnp.float32)
        for cp_v, gmu_v, gsc_v in (
            (cpx_v, gmux_v, gscx_v),
            (cpy_v, gmuy_v, gscy_v),
            (cpz_v, gmuz_v, gscz_v),
        ):
            delta = (cp_v[sl] - gmu_v[sl]) / (jnp.exp(gsc_v[sl]) + 1e-6)
            m2 = m2 + delta * delta
        seed = (lax.bitcast_convert_type(m2, jnp.int32) >> 1) + jnp.int32(
            0x1FBD1DF5
        )
        y = lax.bitcast_convert_type(seed, jnp.float32)
        for _ in range(3):
            y = 0.5 * (y + m2 / y)
        r = y - 1.0
        wv = jnp.clip(w_v[sl], 0.0, 1.0)
        acc = acc + r * r * wv
    acc_v[...] = acc
    pltpu.sync_copy(acc_v, out_hbm.at[wid])


def _sc_residual(positions, scales, nn_idx, contact_points,
                 contact_confidence):
    p = contact_points.shape[0]
    ppw = p // _NW
    mesh = plsc.VectorSubcoreMesh(core_axis_name="c", subcore_axis_name="s")
    f32 = jnp.float32
    run = pl.kernel(
        functools.partial(_sc_residual_body, ppw=ppw),
        out_type=jax.ShapeDtypeStruct((_NW, _L), f32),
        mesh=mesh,
        scratch_types=[
            pltpu.VMEM((ppw,), jnp.int32),
            pltpu.VMEM((ppw,), f32),
            pltpu.VMEM((ppw,), f32),
            pltpu.VMEM((ppw,), f32),
            pltpu.VMEM((ppw,), f32),
            pltpu.VMEM((ppw,), f32),
            pltpu.VMEM((ppw,), f32),
            pltpu.VMEM((ppw,), f32),
            pltpu.VMEM((ppw,), f32),
            pltpu.VMEM((ppw,), f32),
            pltpu.VMEM((ppw,), f32),
            pltpu.VMEM((_L,), f32),
            pltpu.SemaphoreType.DMA,
        ],
    )
    return run(
        positions[:, 0],
        positions[:, 1],
        positions[:, 2],
        scales[:, 0],
        scales[:, 1],
        scales[:, 2],
        nn_idx,
        contact_points[:, 0],
        contact_points[:, 1],
        contact_points[:, 2],
        contact_confidence,
    )


def kernel(positions, scales, contact_points, contact_normals,
           contact_confidence):
    del contact_normals
    n = positions.shape[0]
    p = contact_points.shape[0]

    nn_idx = _nearest_idx(positions, contact_points, tn=512)
    partials = _sc_residual(positions, scales, nn_idx, contact_points,
                            contact_confidence)
    return jnp.sum(partials) / jnp.float32(p)
